# baseline (speedup 1.0000x reference)
import functools

import jax
import jax.numpy as jnp
from jax import lax
from jax.experimental import pallas as pl
from jax.experimental.pallas import tpu as pltpu
from jax.experimental.pallas import tpu_sc as plsc

_HIGH = jax.lax.Precision.HIGHEST

_CHUNK = 128


def _sc_gather_cat(cat, cat_table):
    B = cat.shape[0]
    info = plsc.get_sparse_core_info()
    nc, ns = info.num_cores, info.num_subcores
    nw = nc * ns
    bpw = B // nw
    ec = cat_table.shape[1]
    ct_flat = cat_table.reshape(-1)
    mesh = plsc.VectorSubcoreMesh(core_axis_name="c", subcore_axis_name="s")
    nchunks = bpw // _CHUNK

    @functools.partial(
        pl.kernel,
        mesh=mesh,
        compiler_params=pltpu.CompilerParams(needs_layout_passes=False),
        out_type=jax.ShapeDtypeStruct((B, 128), jnp.float32),
        scratch_types=[
            pltpu.VMEM((bpw,), jnp.int32),
            pltpu.VMEM((ct_flat.shape[0],), jnp.float32),
            pltpu.VMEM((_CHUNK, 128), jnp.float32),
        ],
    )
    def k(cat_hbm, ct_hbm, c_out, cidx, ctab, csel):
        wid = lax.axis_index("s") * nc + lax.axis_index("c")
        base = wid * bpw
        pltpu.sync_copy(cat_hbm.at[pl.ds(base, bpw)], cidx)
        pltpu.sync_copy(ct_hbm, ctab)

        kv16 = jax.lax.iota(jnp.int32, 16)
        for n in range(nchunks):
            for j in range(_CHUNK // 16):
                vidx = cidx[pl.ds(n * _CHUNK + j * 16, 16)]
                fidx = jax.lax.shift_left(vidx, 3)
                kvec = kv16 + j * 16
                for col in range(ec):
                    cv = jnp.full((16,), col, jnp.int32)
                    val = plsc.load_gather(ctab, [fidx + cv])
                    plsc.store_scatter(csel, [kvec, cv], val)
            pltpu.sync_copy(
                csel, c_out.at[pl.ds(base + n * _CHUNK, _CHUNK)])

    return k(cat, ct_flat)


_BLK = 4096


def _mlp_body(u_ref, i_ref, c_ref, d_ref, dwt_ref, db_ref,
              w1u_ref, w1i_ref, w1c_ref, w1d_ref, b1_ref,
              g_ref, bb_ref, w2t_ref, b2_ref, wot_ref, bo_ref,
              o_ref, h_scr, sum_scr, sq_scr, *, batch, nb):
    p = pl.program_id(0)
    b = pl.program_id(1)

    @pl.when(p == 0)
    def _phase_h():
        dd = jnp.maximum(
            jnp.dot(d_ref[...], dwt_ref[...], precision=_HIGH)
            + db_ref[...], 0.0)
        cc = c_ref[...][:, :w1c_ref.shape[0]]
        h = (jnp.dot(u_ref[...], w1u_ref[...], precision=_HIGH)
             + jnp.dot(i_ref[...], w1i_ref[...], precision=_HIGH)
             + jnp.dot(cc, w1c_ref[...], precision=_HIGH)
             + jnp.dot(dd, w1d_ref[...], precision=_HIGH)
             + b1_ref[...])
        h_scr[pl.ds(b * _BLK, _BLK), :] = h
        sum_scr[pl.ds(b, 1), :] = jnp.sum(h, axis=0, keepdims=True)
        sq_scr[pl.ds(b, 1), :] = jnp.sum(h * h, axis=0, keepdims=True)
        o_ref[...] = jnp.zeros_like(o_ref)

    @pl.when(p == 1)
    def _phase_out():
        mean = jnp.sum(sum_scr[...], axis=0, keepdims=True) / batch
        var = jnp.sum(sq_scr[...], axis=0, keepdims=True) / batch - mean * mean
        h = h_scr[pl.ds(b * _BLK, _BLK), :]
        hn = (h - mean) * jax.lax.rsqrt(var + 1e-5) * g_ref[...] + bb_ref[...]
        x = jnp.maximum(hn, 0.0)
        x = jnp.maximum(
            jnp.dot(x, w2t_ref[...], precision=_HIGH) + b2_ref[...], 0.0)
        o_ref[...] = jnp.dot(x, wot_ref[...], precision=_HIGH) + bo_ref[...]


def _tc_mlp(u, i, c, dense, dense_W, dense_b, fc1_W, fc1_b,
            bn_gamma, bn_beta, fc2_W, fc2_b, out_W, out_b):
    B = u.shape[0]
    eu = u.shape[1]
    ec = 8
    cw = c.shape[1]
    nb = B // _BLK
    w1t = fc1_W.T
    hdim = fc1_W.shape[0]

    def rows(bs):
        return pl.BlockSpec((_BLK, bs), lambda p, b: (b * (1 - p), 0))

    def full(shape):
        return pl.BlockSpec(shape, lambda p, b: (0,) * len(shape))

    return pl.pallas_call(
        functools.partial(_mlp_body, batch=float(B), nb=nb),
        grid=(2, nb),
        in_specs=[rows(eu), rows(eu), rows(cw), rows(2),
                  full((2, 8)), full((1, 8)),
                  full((eu, hdim)), full((eu, hdim)), full((ec, hdim)),
                  full((8, hdim)), full((1, hdim)),
                  full((1, hdim)), full((1, hdim)),
                  full((hdim, 32)), full((1, 32)), full((32, 1)),
                  full((1, 1))],
        out_specs=pl.BlockSpec((_BLK, 1), lambda p, b: (b * p, 0)),
        out_shape=jax.ShapeDtypeStruct((B, 1), jnp.float32),
        scratch_shapes=[pltpu.VMEM((B, hdim), jnp.float32),
                        pltpu.VMEM((nb, hdim), jnp.float32),
                        pltpu.VMEM((nb, hdim), jnp.float32)],
    )(u, i, c, dense, dense_W.T, dense_b[None, :],
      w1t[:eu], w1t[eu:2 * eu], w1t[2 * eu:2 * eu + ec], w1t[2 * eu + ec:],
      fc1_b[None, :], bn_gamma[None, :], bn_beta[None, :],
      fc2_W.T, fc2_b[None, :], out_W.T, out_b[None, :])


def kernel(user, item, cat, dense, user_table, item_table, cat_table,
           dense_W, dense_b, fc1_W, fc1_b, bn_gamma, bn_beta,
           fc2_W, fc2_b, out_W, out_b):
    u = jnp.take(user_table, user, axis=0)
    i = jnp.take(item_table, item, axis=0)
    c = _sc_gather_cat(cat.astype(jnp.int32), cat_table)
    return _tc_mlp(u, i, c, dense, dense_W, dense_b, fc1_W, fc1_b,
                   bn_gamma, bn_beta, fc2_W, fc2_b, out_W, out_b)


# speedup vs baseline: 1.0883x; 1.0883x over previous
import functools

import jax
import jax.numpy as jnp
from jax import lax
from jax.experimental import pallas as pl
from jax.experimental.pallas import tpu as pltpu
from jax.experimental.pallas import tpu_sc as plsc

_HIGH = jax.lax.Precision.HIGHEST

_CHUNK = 128


def _sc_gather_cat(cat, cat_table):
    B = cat.shape[0]
    info = plsc.get_sparse_core_info()
    nc, ns = info.num_cores, info.num_subcores
    nw = nc * ns
    bpw = B // nw
    ec = cat_table.shape[1]
    ct_flat = cat_table.reshape(-1)
    mesh = plsc.VectorSubcoreMesh(core_axis_name="c", subcore_axis_name="s")
    nchunks = bpw // _CHUNK

    @functools.partial(
        pl.kernel,
        mesh=mesh,
        compiler_params=pltpu.CompilerParams(needs_layout_passes=False),
        out_type=jax.ShapeDtypeStruct((B, 128), jnp.float32),
        scratch_types=[
            pltpu.VMEM((bpw,), jnp.int32),
            pltpu.VMEM((ct_flat.shape[0],), jnp.float32),
            pltpu.VMEM((_CHUNK, 128), jnp.float32),
        ],
    )
    def k(cat_hbm, ct_hbm, c_out, cidx, ctab, csel):
        wid = lax.axis_index("s") * nc + lax.axis_index("c")
        base = wid * bpw
        pltpu.sync_copy(cat_hbm.at[pl.ds(base, bpw)], cidx)
        pltpu.sync_copy(ct_hbm, ctab)

        kv16 = jax.lax.iota(jnp.int32, 16)
        for n in range(nchunks):
            for j in range(_CHUNK // 16):
                vidx = cidx[pl.ds(n * _CHUNK + j * 16, 16)]
                fidx = jax.lax.shift_left(vidx, 3)
                kvec = kv16 + j * 16
                for col in range(ec):
                    cv = jnp.full((16,), col, jnp.int32)
                    val = plsc.load_gather(ctab, [fidx + cv])
                    plsc.store_scatter(csel, [kvec, cv], val)
            pltpu.sync_copy(
                csel, c_out.at[pl.ds(base + n * _CHUNK, _CHUNK)])

    return k(cat, ct_flat)


_BLK = 2048


def _mlp_body(u_ref, i_ref, c_ref, d_ref, dwt_ref, db_ref,
              w1u_ref, w1i_ref, w1c_ref, w1d_ref, b1_ref,
              g_ref, bb_ref, w2t_ref, b2_ref, wot_ref, bo_ref,
              o_ref, h_scr, sum_scr, sq_scr, *, batch, nb):
    p = pl.program_id(0)
    b = pl.program_id(1)

    @pl.when(p == 0)
    def _phase_h():
        dd = jnp.maximum(
            jnp.dot(d_ref[...], dwt_ref[...], precision=_HIGH)
            + db_ref[...], 0.0)
        cc = c_ref[...][:, :w1c_ref.shape[0]]
        h = (jnp.dot(u_ref[...], w1u_ref[...], precision=_HIGH)
             + jnp.dot(i_ref[...], w1i_ref[...], precision=_HIGH)
             + jnp.dot(cc, w1c_ref[...], precision=_HIGH)
             + jnp.dot(dd, w1d_ref[...], precision=_HIGH)
             + b1_ref[...])
        h_scr[pl.ds(b * _BLK, _BLK), :] = h
        sum_scr[pl.ds(b, 1), :] = jnp.sum(h, axis=0, keepdims=True)
        sq_scr[pl.ds(b, 1), :] = jnp.sum(h * h, axis=0, keepdims=True)
        o_ref[...] = jnp.zeros_like(o_ref)

    @pl.when(p == 1)
    def _phase_out():
        mean = jnp.sum(sum_scr[...], axis=0, keepdims=True) / batch
        var = jnp.sum(sq_scr[...], axis=0, keepdims=True) / batch - mean * mean
        h = h_scr[pl.ds(b * _BLK, _BLK), :]
        hn = (h - mean) * jax.lax.rsqrt(var + 1e-5) * g_ref[...] + bb_ref[...]
        x = jnp.maximum(hn, 0.0)
        x = jnp.maximum(
            jnp.dot(x, w2t_ref[...], precision=_HIGH) + b2_ref[...], 0.0)
        o_ref[...] = jnp.dot(x, wot_ref[...], precision=_HIGH) + bo_ref[...]


def _tc_mlp(u, i, c, dense, dense_W, dense_b, fc1_W, fc1_b,
            bn_gamma, bn_beta, fc2_W, fc2_b, out_W, out_b):
    B = u.shape[0]
    eu = u.shape[1]
    ec = 8
    cw = c.shape[1]
    nb = B // _BLK
    w1t = fc1_W.T
    hdim = fc1_W.shape[0]

    def rows(bs):
        return pl.BlockSpec((_BLK, bs), lambda p, b: (b * (1 - p), 0))

    def full(shape):
        return pl.BlockSpec(shape, lambda p, b: (0,) * len(shape))

    return pl.pallas_call(
        functools.partial(_mlp_body, batch=float(B), nb=nb),
        grid=(2, nb),
        in_specs=[rows(eu), rows(eu), rows(cw), rows(2),
                  full((2, 8)), full((1, 8)),
                  full((eu, hdim)), full((eu, hdim)), full((ec, hdim)),
                  full((8, hdim)), full((1, hdim)),
                  full((1, hdim)), full((1, hdim)),
                  full((hdim, 32)), full((1, 32)), full((32, 1)),
                  full((1, 1))],
        out_specs=pl.BlockSpec((_BLK, 1), lambda p, b: (b * p, 0)),
        out_shape=jax.ShapeDtypeStruct((B, 1), jnp.float32),
        scratch_shapes=[pltpu.VMEM((B, hdim), jnp.float32),
                        pltpu.VMEM((nb, hdim), jnp.float32),
                        pltpu.VMEM((nb, hdim), jnp.float32)],
    )(u, i, c, dense, dense_W.T, dense_b[None, :],
      w1t[:eu], w1t[eu:2 * eu], w1t[2 * eu:2 * eu + ec], w1t[2 * eu + ec:],
      fc1_b[None, :], bn_gamma[None, :], bn_beta[None, :],
      fc2_W.T, fc2_b[None, :], out_W.T, out_b[None, :])


def kernel(user, item, cat, dense, user_table, item_table, cat_table,
           dense_W, dense_b, fc1_W, fc1_b, bn_gamma, bn_beta,
           fc2_W, fc2_b, out_W, out_b):
    u = jnp.take(user_table, user, axis=0, mode="clip")
    i = jnp.take(item_table, item, axis=0, mode="clip")
    c = _sc_gather_cat(cat.astype(jnp.int32), cat_table)
    return _tc_mlp(u, i, c, dense, dense_W, dense_b, fc1_W, fc1_b,
                   bn_gamma, bn_beta, fc2_W, fc2_b, out_W, out_b)


# speedup vs baseline: 1.0918x; 1.0032x over previous
import functools

import jax
import jax.numpy as jnp
from jax import lax
from jax.experimental import pallas as pl
from jax.experimental.pallas import tpu as pltpu
from jax.experimental.pallas import tpu_sc as plsc

_HIGH = jax.lax.Precision.HIGHEST

_CHUNK = 128


def _sc_gather_cat(cat, cat_table):
    B = cat.shape[0]
    info = plsc.get_sparse_core_info()
    nc, ns = info.num_cores, info.num_subcores
    nw = nc * ns
    bpw = B // nw
    ec = cat_table.shape[1]
    ct_flat = cat_table.reshape(-1)
    mesh = plsc.VectorSubcoreMesh(core_axis_name="c", subcore_axis_name="s")
    nchunks = bpw // _CHUNK

    @functools.partial(
        pl.kernel,
        mesh=mesh,
        compiler_params=pltpu.CompilerParams(needs_layout_passes=False),
        out_type=jax.ShapeDtypeStruct((B, 128), jnp.float32),
        scratch_types=[
            pltpu.VMEM((bpw,), jnp.int32),
            pltpu.VMEM((ct_flat.shape[0],), jnp.float32),
            pltpu.VMEM((bpw, 128), jnp.float32),
        ],
    )
    def k(cat_hbm, ct_hbm, c_out, cidx, ctab, csel):
        wid = lax.axis_index("s") * nc + lax.axis_index("c")
        base = wid * bpw
        pltpu.sync_copy(cat_hbm.at[pl.ds(base, bpw)], cidx)
        pltpu.sync_copy(ct_hbm, ctab)

        kv16 = jax.lax.iota(jnp.int32, 16)
        for j in range(bpw // 16):
            vidx = cidx[pl.ds(j * 16, 16)]
            fidx = jax.lax.shift_left(vidx, 3)
            kvec = kv16 + j * 16
            for col in range(ec):
                cv = jnp.full((16,), col, jnp.int32)
                val = plsc.load_gather(ctab, [fidx + cv])
                plsc.store_scatter(csel, [kvec, cv], val)
        pltpu.sync_copy(csel, c_out.at[pl.ds(base, bpw)])

    return k(cat, ct_flat)


_BLK = 2048


def _mlp_body(u_ref, i_ref, c_ref, d_ref, dwt_ref, db_ref,
              w1u_ref, w1i_ref, w1c_ref, w1d_ref, b1_ref,
              g_ref, bb_ref, w2t_ref, b2_ref, wot_ref, bo_ref,
              o_ref, h_scr, sum_scr, sq_scr, *, batch, nb):
    p = pl.program_id(0)
    b = pl.program_id(1)

    @pl.when(p == 0)
    def _phase_h():
        dd = jnp.maximum(
            jnp.dot(d_ref[...], dwt_ref[...], precision=_HIGH)
            + db_ref[...], 0.0)
        cc = c_ref[...][:, :w1c_ref.shape[0]]
        h = (jnp.dot(u_ref[...], w1u_ref[...], precision=_HIGH)
             + jnp.dot(i_ref[...], w1i_ref[...], precision=_HIGH)
             + jnp.dot(cc, w1c_ref[...], precision=_HIGH)
             + jnp.dot(dd, w1d_ref[...], precision=_HIGH)
             + b1_ref[...])
        h_scr[pl.ds(b * _BLK, _BLK), :] = h
        sum_scr[pl.ds(b, 1), :] = jnp.sum(h, axis=0, keepdims=True)
        sq_scr[pl.ds(b, 1), :] = jnp.sum(h * h, axis=0, keepdims=True)
        o_ref[...] = jnp.zeros_like(o_ref)

    @pl.when(p == 1)
    def _phase_out():
        mean = jnp.sum(sum_scr[...], axis=0, keepdims=True) / batch
        var = jnp.sum(sq_scr[...], axis=0, keepdims=True) / batch - mean * mean
        h = h_scr[pl.ds(b * _BLK, _BLK), :]
        hn = (h - mean) * jax.lax.rsqrt(var + 1e-5) * g_ref[...] + bb_ref[...]
        x = jnp.maximum(hn, 0.0)
        x = jnp.maximum(
            jnp.dot(x, w2t_ref[...], precision=_HIGH) + b2_ref[...], 0.0)
        o_ref[...] = jnp.dot(x, wot_ref[...], precision=_HIGH) + bo_ref[...]


def _tc_mlp(u, i, c, dense, dense_W, dense_b, fc1_W, fc1_b,
            bn_gamma, bn_beta, fc2_W, fc2_b, out_W, out_b):
    B = u.shape[0]
    eu = u.shape[1]
    ec = 8
    cw = c.shape[1]
    nb = B // _BLK
    w1t = fc1_W.T
    hdim = fc1_W.shape[0]

    def rows(bs):
        return pl.BlockSpec((_BLK, bs), lambda p, b: (b * (1 - p), 0))

    def full(shape):
        return pl.BlockSpec(shape, lambda p, b: (0,) * len(shape))

    return pl.pallas_call(
        functools.partial(_mlp_body, batch=float(B), nb=nb),
        grid=(2, nb),
        in_specs=[rows(eu), rows(eu), rows(cw), rows(2),
                  full((2, 8)), full((1, 8)),
                  full((eu, hdim)), full((eu, hdim)), full((ec, hdim)),
                  full((8, hdim)), full((1, hdim)),
                  full((1, hdim)), full((1, hdim)),
                  full((hdim, 32)), full((1, 32)), full((32, 1)),
                  full((1, 1))],
        out_specs=pl.BlockSpec((_BLK, 1), lambda p, b: (b * p, 0)),
        out_shape=jax.ShapeDtypeStruct((B, 1), jnp.float32),
        scratch_shapes=[pltpu.VMEM((B, hdim), jnp.float32),
                        pltpu.VMEM((nb, hdim), jnp.float32),
                        pltpu.VMEM((nb, hdim), jnp.float32)],
    )(u, i, c, dense, dense_W.T, dense_b[None, :],
      w1t[:eu], w1t[eu:2 * eu], w1t[2 * eu:2 * eu + ec], w1t[2 * eu + ec:],
      fc1_b[None, :], bn_gamma[None, :], bn_beta[None, :],
      fc2_W.T, fc2_b[None, :], out_W.T, out_b[None, :])


def kernel(user, item, cat, dense, user_table, item_table, cat_table,
           dense_W, dense_b, fc1_W, fc1_b, bn_gamma, bn_beta,
           fc2_W, fc2_b, out_W, out_b):
    u = jnp.take(user_table, user, axis=0, mode="clip")
    i = jnp.take(item_table, item, axis=0, mode="clip")
    c = _sc_gather_cat(cat.astype(jnp.int32), cat_table)
    return _tc_mlp(u, i, c, dense, dense_W, dense_b, fc1_W, fc1_b,
                   bn_gamma, bn_beta, fc2_W, fc2_b, out_W, out_b)


# speedup vs baseline: 1.5965x; 1.4622x over previous
import functools

import jax
import jax.numpy as jnp
from jax import lax
from jax.experimental import pallas as pl
from jax.experimental.pallas import tpu as pltpu
from jax.experimental.pallas import tpu_sc as plsc

_HIGH = jax.lax.Precision.DEFAULT

_CHUNK = 128


def _sc_gather_cat(cat, cat_table):
    B = cat.shape[0]
    info = plsc.get_sparse_core_info()
    nc, ns = info.num_cores, info.num_subcores
    nw = nc * ns
    bpw = B // nw
    ec = cat_table.shape[1]
    ct_flat = cat_table.reshape(-1)
    mesh = plsc.VectorSubcoreMesh(core_axis_name="c", subcore_axis_name="s")
    nchunks = bpw // _CHUNK

    @functools.partial(
        pl.kernel,
        mesh=mesh,
        compiler_params=pltpu.CompilerParams(needs_layout_passes=False),
        out_type=jax.ShapeDtypeStruct((B, 128), jnp.float32),
        scratch_types=[
            pltpu.VMEM((bpw,), jnp.int32),
            pltpu.VMEM((ct_flat.shape[0],), jnp.float32),
            pltpu.VMEM((bpw, 128), jnp.float32),
        ],
    )
    def k(cat_hbm, ct_hbm, c_out, cidx, ctab, csel):
        wid = lax.axis_index("s") * nc + lax.axis_index("c")
        base = wid * bpw
        pltpu.sync_copy(cat_hbm.at[pl.ds(base, bpw)], cidx)
        pltpu.sync_copy(ct_hbm, ctab)

        kv16 = jax.lax.iota(jnp.int32, 16)
        for j in range(bpw // 16):
            vidx = cidx[pl.ds(j * 16, 16)]
            fidx = jax.lax.shift_left(vidx, 3)
            kvec = kv16 + j * 16
            for col in range(ec):
                cv = jnp.full((16,), col, jnp.int32)
                val = plsc.load_gather(ctab, [fidx + cv])
                plsc.store_scatter(csel, [kvec, cv], val)
        pltpu.sync_copy(csel, c_out.at[pl.ds(base, bpw)])

    return k(cat, ct_flat)


_BLK = 2048


def _mlp_body(u_ref, i_ref, c_ref, d_ref, dwt_ref, db_ref,
              w1u_ref, w1i_ref, w1c_ref, w1d_ref, b1_ref,
              g_ref, bb_ref, w2t_ref, b2_ref, wot_ref, bo_ref,
              o_ref, h_scr, sum_scr, sq_scr, *, batch, nb):
    p = pl.program_id(0)
    b = pl.program_id(1)

    @pl.when(p == 0)
    def _phase_h():
        dd = jnp.maximum(
            jnp.dot(d_ref[...], dwt_ref[...], precision=_HIGH)
            + db_ref[...], 0.0)
        cc = c_ref[...][:, :w1c_ref.shape[0]]
        h = (jnp.dot(u_ref[...], w1u_ref[...], precision=_HIGH)
             + jnp.dot(i_ref[...], w1i_ref[...], precision=_HIGH)
             + jnp.dot(cc, w1c_ref[...], precision=_HIGH)
             + jnp.dot(dd, w1d_ref[...], precision=_HIGH)
             + b1_ref[...])
        h_scr[pl.ds(b * _BLK, _BLK), :] = h
        sum_scr[pl.ds(b, 1), :] = jnp.sum(h, axis=0, keepdims=True)
        sq_scr[pl.ds(b, 1), :] = jnp.sum(h * h, axis=0, keepdims=True)
        o_ref[...] = jnp.zeros_like(o_ref)

    @pl.when(p == 1)
    def _phase_out():
        mean = jnp.sum(sum_scr[...], axis=0, keepdims=True) / batch
        var = jnp.sum(sq_scr[...], axis=0, keepdims=True) / batch - mean * mean
        h = h_scr[pl.ds(b * _BLK, _BLK), :]
        hn = (h - mean) * jax.lax.rsqrt(var + 1e-5) * g_ref[...] + bb_ref[...]
        x = jnp.maximum(hn, 0.0)
        x = jnp.maximum(
            jnp.dot(x, w2t_ref[...], precision=_HIGH) + b2_ref[...], 0.0)
        o_ref[...] = jnp.dot(x, wot_ref[...], precision=_HIGH) + bo_ref[...]


def _tc_mlp(u, i, c, dense, dense_W, dense_b, fc1_W, fc1_b,
            bn_gamma, bn_beta, fc2_W, fc2_b, out_W, out_b):
    B = u.shape[0]
    eu = u.shape[1]
    ec = 8
    cw = c.shape[1]
    nb = B // _BLK
    w1t = fc1_W.T
    hdim = fc1_W.shape[0]

    def rows(bs):
        return pl.BlockSpec((_BLK, bs), lambda p, b: (b * (1 - p), 0))

    def full(shape):
        return pl.BlockSpec(shape, lambda p, b: (0,) * len(shape))

    return pl.pallas_call(
        functools.partial(_mlp_body, batch=float(B), nb=nb),
        grid=(2, nb),
        in_specs=[rows(eu), rows(eu), rows(cw), rows(2),
                  full((2, 8)), full((1, 8)),
                  full((eu, hdim)), full((eu, hdim)), full((ec, hdim)),
                  full((8, hdim)), full((1, hdim)),
                  full((1, hdim)), full((1, hdim)),
                  full((hdim, 32)), full((1, 32)), full((32, 1)),
                  full((1, 1))],
        out_specs=pl.BlockSpec((_BLK, 1), lambda p, b: (b * p, 0)),
        out_shape=jax.ShapeDtypeStruct((B, 1), jnp.float32),
        scratch_shapes=[pltpu.VMEM((B, hdim), jnp.float32),
                        pltpu.VMEM((nb, hdim), jnp.float32),
                        pltpu.VMEM((nb, hdim), jnp.float32)],
    )(u, i, c, dense, dense_W.T, dense_b[None, :],
      w1t[:eu], w1t[eu:2 * eu], w1t[2 * eu:2 * eu + ec], w1t[2 * eu + ec:],
      fc1_b[None, :], bn_gamma[None, :], bn_beta[None, :],
      fc2_W.T, fc2_b[None, :], out_W.T, out_b[None, :])


def kernel(user, item, cat, dense, user_table, item_table, cat_table,
           dense_W, dense_b, fc1_W, fc1_b, bn_gamma, bn_beta,
           fc2_W, fc2_b, out_W, out_b):
    u = jnp.take(user_table, user, axis=0, mode="clip")
    i = jnp.take(item_table, item, axis=0, mode="clip")
    c = _sc_gather_cat(cat.astype(jnp.int32), cat_table)
    return _tc_mlp(u, i, c, dense, dense_W, dense_b, fc1_W, fc1_b,
                   bn_gamma, bn_beta, fc2_W, fc2_b, out_W, out_b)


# speedup vs baseline: 1.5978x; 1.0009x over previous
import functools

import jax
import jax.numpy as jnp
from jax import lax
from jax.experimental import pallas as pl
from jax.experimental.pallas import tpu as pltpu
from jax.experimental.pallas import tpu_sc as plsc

_HIGH = jax.lax.Precision.DEFAULT

_CHUNK = 128


def _sc_gather_cat(cat, cat_table):
    B = cat.shape[0]
    info = plsc.get_sparse_core_info()
    nc, ns = info.num_cores, info.num_subcores
    nw = nc * ns
    bpw = B // nw
    ec = cat_table.shape[1]
    ct_flat = cat_table.reshape(-1)
    mesh = plsc.VectorSubcoreMesh(core_axis_name="c", subcore_axis_name="s")
    nchunks = bpw // _CHUNK

    @functools.partial(
        pl.kernel,
        mesh=mesh,
        compiler_params=pltpu.CompilerParams(needs_layout_passes=False),
        out_type=jax.ShapeDtypeStruct((B, 128), jnp.float32),
        scratch_types=[
            pltpu.VMEM((bpw,), jnp.int32),
            pltpu.VMEM((ct_flat.shape[0],), jnp.float32),
            pltpu.VMEM((bpw, 128), jnp.float32),
        ],
    )
    def k(cat_hbm, ct_hbm, c_out, cidx, ctab, csel):
        wid = lax.axis_index("s") * nc + lax.axis_index("c")
        base = wid * bpw
        pltpu.sync_copy(cat_hbm.at[pl.ds(base, bpw)], cidx)
        pltpu.sync_copy(ct_hbm, ctab)

        kv16 = jax.lax.iota(jnp.int32, 16)
        for j in range(bpw // 16):
            vidx = cidx[pl.ds(j * 16, 16)]
            fidx = jax.lax.shift_left(vidx, 3)
            kvec = kv16 + j * 16
            for col in range(ec):
                cv = jnp.full((16,), col, jnp.int32)
                val = plsc.load_gather(ctab, [fidx + cv])
                plsc.store_scatter(csel, [kvec, cv], val)
        pltpu.sync_copy(csel, c_out.at[pl.ds(base, bpw)])

    return k(cat, ct_flat)


_BLK = 2048


def _dot_t(x, w_ref):
    return jax.lax.dot_general(
        x, w_ref[...], (((1,), (1,)), ((), ())), precision=_HIGH)


def _mlp_body(u_ref, i_ref, c_ref, d_ref, dw_ref, db_ref,
              w1_ref, b1_ref,
              g_ref, bb_ref, w2_ref, b2_ref, wo_ref, bo_ref,
              o_ref, h_scr, sum_scr, sq_scr, *, batch, eu, ec):
    p = pl.program_id(0)
    b = pl.program_id(1)

    @pl.when(p == 0)
    def _phase_h():
        dd = jnp.maximum(_dot_t(d_ref[...], dw_ref) + db_ref[...], 0.0)
        cc = c_ref[...][:, :ec]
        h = (_dot_t(u_ref[...], w1_ref.at[:, :eu])
             + _dot_t(i_ref[...], w1_ref.at[:, eu:2 * eu])
             + _dot_t(cc, w1_ref.at[:, 2 * eu:2 * eu + ec])
             + _dot_t(dd, w1_ref.at[:, 2 * eu + ec:])
             + b1_ref[...])
        h_scr[pl.ds(b * _BLK, _BLK), :] = h
        sum_scr[pl.ds(b, 1), :] = jnp.sum(h, axis=0, keepdims=True)
        sq_scr[pl.ds(b, 1), :] = jnp.sum(h * h, axis=0, keepdims=True)
        o_ref[...] = jnp.zeros_like(o_ref)

    @pl.when(p == 1)
    def _phase_out():
        mean = jnp.sum(sum_scr[...], axis=0, keepdims=True) / batch
        var = jnp.sum(sq_scr[...], axis=0, keepdims=True) / batch - mean * mean
        h = h_scr[pl.ds(b * _BLK, _BLK), :]
        hn = (h - mean) * jax.lax.rsqrt(var + 1e-5) * g_ref[...] + bb_ref[...]
        x = jnp.maximum(hn, 0.0)
        x = jnp.maximum(_dot_t(x, w2_ref) + b2_ref[...], 0.0)
        o_ref[...] = jnp.dot(x, wo_ref[...], precision=_HIGH) + bo_ref[...]


def _tc_mlp(u, i, c, dense, dense_W, dense_b, fc1_W, fc1_b,
            bn_gamma, bn_beta, fc2_W, fc2_b, out_W, out_b):
    B = u.shape[0]
    eu = u.shape[1]
    ec = 8
    cw = c.shape[1]
    nb = B // _BLK
    hdim = fc1_W.shape[0]

    def rows(bs):
        return pl.BlockSpec((_BLK, bs), lambda p, b: (b * (1 - p), 0))

    def full(shape):
        return pl.BlockSpec(shape, lambda p, b: (0,) * len(shape))

    return pl.pallas_call(
        functools.partial(_mlp_body, batch=float(B), eu=eu, ec=ec),
        grid=(2, nb),
        in_specs=[rows(eu), rows(eu), rows(cw), rows(2),
                  full((8, 2)), full((1, 8)),
                  full((hdim, 2 * eu + ec + 8)), full((1, hdim)),
                  full((1, hdim)), full((1, hdim)),
                  full((32, hdim)), full((1, 32)), full((32, 1)),
                  full((1, 1))],
        out_specs=pl.BlockSpec((_BLK, 1), lambda p, b: (b * p, 0)),
        out_shape=jax.ShapeDtypeStruct((B, 1), jnp.float32),
        scratch_shapes=[pltpu.VMEM((B, hdim), jnp.float32),
                        pltpu.VMEM((nb, hdim), jnp.float32),
                        pltpu.VMEM((nb, hdim), jnp.float32)],
    )(u, i, c, dense, dense_W, dense_b[None, :],
      fc1_W, fc1_b[None, :], bn_gamma[None, :], bn_beta[None, :],
      fc2_W, fc2_b[None, :], out_W.T, out_b[None, :])


def kernel(user, item, cat, dense, user_table, item_table, cat_table,
           dense_W, dense_b, fc1_W, fc1_b, bn_gamma, bn_beta,
           fc2_W, fc2_b, out_W, out_b):
    u = jnp.take(user_table, user, axis=0, mode="clip")
    i = jnp.take(item_table, item, axis=0, mode="clip")
    c = _sc_gather_cat(cat.astype(jnp.int32), cat_table)
    return _tc_mlp(u, i, c, dense, dense_W, dense_b, fc1_W, fc1_b,
                   bn_gamma, bn_beta, fc2_W, fc2_b, out_W, out_b)


# speedup vs baseline: 1.6298x; 1.0200x over previous
import functools

import jax
import jax.numpy as jnp
from jax import lax
from jax.experimental import pallas as pl
from jax.experimental.pallas import tpu as pltpu
from jax.experimental.pallas import tpu_sc as plsc

_HIGH = jax.lax.Precision.DEFAULT

_CHUNK = 128


def _sc_gather_cat(cat, cat_table):
    B = cat.shape[0]
    info = plsc.get_sparse_core_info()
    nc, ns = info.num_cores, info.num_subcores
    nw = nc * ns
    bpw = B // nw
    ec = cat_table.shape[1]
    ct_flat = cat_table.reshape(-1)
    mesh = plsc.VectorSubcoreMesh(core_axis_name="c", subcore_axis_name="s")
    nchunks = bpw // _CHUNK

    @functools.partial(
        pl.kernel,
        mesh=mesh,
        compiler_params=pltpu.CompilerParams(needs_layout_passes=False),
        out_type=jax.ShapeDtypeStruct((B, 128), jnp.float32),
        scratch_types=[
            pltpu.VMEM((bpw,), jnp.int32),
            pltpu.VMEM((ct_flat.shape[0],), jnp.float32),
            pltpu.VMEM((bpw, 128), jnp.float32),
            pltpu.SemaphoreType.DMA,
        ],
    )
    def k(cat_hbm, ct_hbm, c_out, cidx, ctab, csel, sem):
        wid = lax.axis_index("s") * nc + lax.axis_index("c")
        base = wid * bpw
        pltpu.sync_copy(cat_hbm.at[pl.ds(base, bpw)], cidx)
        pltpu.sync_copy(ct_hbm, ctab)

        kv16 = jax.lax.iota(jnp.int32, 16)
        copies = []
        for n in range(nchunks):
            for j in range(_CHUNK // 16):
                vidx = cidx[pl.ds(n * _CHUNK + j * 16, 16)]
                fidx = jax.lax.shift_left(vidx, 3)
                kvec = kv16 + n * _CHUNK + j * 16
                for col in range(ec):
                    cv = jnp.full((16,), col, jnp.int32)
                    val = plsc.load_gather(ctab, [fidx + cv])
                    plsc.store_scatter(csel, [kvec, cv], val)
            copies.append(pltpu.async_copy(
                csel.at[pl.ds(n * _CHUNK, _CHUNK)],
                c_out.at[pl.ds(base + n * _CHUNK, _CHUNK)], sem))
        for cp in copies:
            cp.wait()

    return k(cat, ct_flat)


_BLK = 2048


def _dot_t(x, w_ref):
    return jax.lax.dot_general(
        x, w_ref[...], (((1,), (1,)), ((), ())), precision=_HIGH)


def _mlp_body(u_ref, i_ref, c_ref, d_ref, dw_ref, db_ref,
              w1_ref, b1_ref,
              g_ref, bb_ref, w2_ref, b2_ref, wo_ref, bo_ref,
              o_ref, h_scr, sum_scr, sq_scr, *, batch, eu, ec):
    p = pl.program_id(0)
    b = pl.program_id(1)

    @pl.when(p == 0)
    def _phase_h():
        dd = jnp.maximum(_dot_t(d_ref[...], dw_ref) + db_ref[...], 0.0)
        cc = c_ref[...][:, :ec]
        vec = jnp.concatenate([u_ref[...], i_ref[...], cc, dd], axis=1)
        h = _dot_t(vec, w1_ref) + b1_ref[...]
        h_scr[pl.ds(b * _BLK, _BLK), :] = h
        sum_scr[pl.ds(b, 1), :] = jnp.sum(h, axis=0, keepdims=True)
        sq_scr[pl.ds(b, 1), :] = jnp.sum(h * h, axis=0, keepdims=True)
        o_ref[...] = jnp.zeros_like(o_ref)

    @pl.when(p == 1)
    def _phase_out():
        mean = jnp.sum(sum_scr[...], axis=0, keepdims=True) / batch
        var = jnp.sum(sq_scr[...], axis=0, keepdims=True) / batch - mean * mean
        h = h_scr[pl.ds(b * _BLK, _BLK), :]
        hn = (h - mean) * jax.lax.rsqrt(var + 1e-5) * g_ref[...] + bb_ref[...]
        x = jnp.maximum(hn, 0.0)
        x = jnp.maximum(_dot_t(x, w2_ref) + b2_ref[...], 0.0)
        o_ref[...] = jnp.dot(x, wo_ref[...], precision=_HIGH) + bo_ref[...]


def _tc_mlp(u, i, c, dense, dense_W, dense_b, fc1_W, fc1_b,
            bn_gamma, bn_beta, fc2_W, fc2_b, out_W, out_b):
    B = u.shape[0]
    eu = u.shape[1]
    ec = 8
    cw = c.shape[1]
    nb = B // _BLK
    hdim = fc1_W.shape[0]

    def rows(bs):
        return pl.BlockSpec((_BLK, bs), lambda p, b: (b * (1 - p), 0))

    def full(shape):
        return pl.BlockSpec(shape, lambda p, b: (0,) * len(shape))

    return pl.pallas_call(
        functools.partial(_mlp_body, batch=float(B), eu=eu, ec=ec),
        grid=(2, nb),
        in_specs=[rows(eu), rows(eu), rows(cw), rows(2),
                  full((8, 2)), full((1, 8)),
                  full((hdim, 2 * eu + ec + 8)), full((1, hdim)),
                  full((1, hdim)), full((1, hdim)),
                  full((32, hdim)), full((1, 32)), full((32, 1)),
                  full((1, 1))],
        out_specs=pl.BlockSpec((_BLK, 1), lambda p, b: (b * p, 0)),
        out_shape=jax.ShapeDtypeStruct((B, 1), jnp.float32),
        scratch_shapes=[pltpu.VMEM((B, hdim), jnp.float32),
                        pltpu.VMEM((nb, hdim), jnp.float32),
                        pltpu.VMEM((nb, hdim), jnp.float32)],
    )(u, i, c, dense, dense_W, dense_b[None, :],
      fc1_W, fc1_b[None, :], bn_gamma[None, :], bn_beta[None, :],
      fc2_W, fc2_b[None, :], out_W.T, out_b[None, :])


def kernel(user, item, cat, dense, user_table, item_table, cat_table,
           dense_W, dense_b, fc1_W, fc1_b, bn_gamma, bn_beta,
           fc2_W, fc2_b, out_W, out_b):
    u = jnp.take(user_table, user, axis=0, mode="clip")
    i = jnp.take(item_table, item, axis=0, mode="clip")
    c = _sc_gather_cat(cat.astype(jnp.int32), cat_table)
    return _tc_mlp(u, i, c, dense, dense_W, dense_b, fc1_W, fc1_b,
                   bn_gamma, bn_beta, fc2_W, fc2_b, out_W, out_b)


# speedup vs baseline: 1.6423x; 1.0077x over previous
import functools

import jax
import jax.numpy as jnp
from jax import lax
from jax.experimental import pallas as pl
from jax.experimental.pallas import tpu as pltpu
from jax.experimental.pallas import tpu_sc as plsc

_HIGH = jax.lax.Precision.DEFAULT

_CHUNK = 128


def _sc_gather_cat(cat, cat_table):
    B = cat.shape[0]
    info = plsc.get_sparse_core_info()
    nc, ns = info.num_cores, info.num_subcores
    nw = nc * ns
    bpw = B // nw
    ec = cat_table.shape[1]
    ct_flat = cat_table.reshape(-1)
    mesh = plsc.VectorSubcoreMesh(core_axis_name="c", subcore_axis_name="s")
    nchunks = bpw // _CHUNK

    @functools.partial(
        pl.kernel,
        mesh=mesh,
        compiler_params=pltpu.CompilerParams(needs_layout_passes=False),
        out_type=jax.ShapeDtypeStruct((B, 128), jnp.float32),
        scratch_types=[
            pltpu.VMEM((bpw,), jnp.int32),
            pltpu.VMEM((ct_flat.shape[0],), jnp.float32),
            pltpu.VMEM((bpw, 128), jnp.float32),
            pltpu.SemaphoreType.DMA,
        ],
    )
    def k(cat_hbm, ct_hbm, c_out, cidx, ctab, csel, sem):
        wid = lax.axis_index("s") * nc + lax.axis_index("c")
        base = wid * bpw
        c1 = pltpu.async_copy(cat_hbm.at[pl.ds(base, bpw)], cidx, sem)
        c2 = pltpu.async_copy(ct_hbm, ctab, sem)
        c1.wait()
        c2.wait()

        kv16 = jax.lax.iota(jnp.int32, 16)
        copies = []
        for n in range(nchunks):
            for j in range(_CHUNK // 16):
                vidx = cidx[pl.ds(n * _CHUNK + j * 16, 16)]
                fidx = jax.lax.shift_left(vidx, 3)
                kvec = kv16 + n * _CHUNK + j * 16
                for col in range(ec):
                    cv = jnp.full((16,), col, jnp.int32)
                    val = plsc.load_gather(ctab, [fidx + cv])
                    plsc.store_scatter(csel, [kvec, cv], val)
            copies.append(pltpu.async_copy(
                csel.at[pl.ds(n * _CHUNK, _CHUNK)],
                c_out.at[pl.ds(base + n * _CHUNK, _CHUNK)], sem))
        for cp in copies:
            cp.wait()

    return k(cat, ct_flat)


_BLK = 2048


def _dot_t(x, w_ref):
    return jax.lax.dot_general(
        x, w_ref[...], (((1,), (1,)), ((), ())), precision=_HIGH)


def _mlp_body(u_ref, i_ref, c_ref, d_ref, dw_ref, db_ref,
              w1_ref, b1_ref,
              g_ref, bb_ref, w2_ref, b2_ref, wo_ref, bo_ref,
              o_ref, h_scr, sum_scr, sq_scr, *, batch, eu, ec):
    p = pl.program_id(0)
    b = pl.program_id(1)

    @pl.when(p == 0)
    def _phase_h():
        dd = jnp.maximum(_dot_t(d_ref[...], dw_ref) + db_ref[...], 0.0)
        cc = c_ref[...][:, :ec]
        vec = jnp.concatenate([u_ref[...], i_ref[...], cc, dd], axis=1)
        h = _dot_t(vec, w1_ref) + b1_ref[...]
        h_scr[pl.ds(b * _BLK, _BLK), :] = h
        sum_scr[pl.ds(b, 1), :] = jnp.sum(h, axis=0, keepdims=True)
        sq_scr[pl.ds(b, 1), :] = jnp.sum(h * h, axis=0, keepdims=True)
        o_ref[...] = jnp.zeros_like(o_ref)

    @pl.when(p == 1)
    def _phase_out():
        mean = jnp.sum(sum_scr[...], axis=0, keepdims=True) / batch
        var = jnp.sum(sq_scr[...], axis=0, keepdims=True) / batch - mean * mean
        h = h_scr[pl.ds(b * _BLK, _BLK), :]
        hn = (h - mean) * jax.lax.rsqrt(var + 1e-5) * g_ref[...] + bb_ref[...]
        x = jnp.maximum(hn, 0.0)
        x = jnp.maximum(_dot_t(x, w2_ref) + b2_ref[...], 0.0)
        o_ref[...] = jnp.dot(x, wo_ref[...], precision=_HIGH) + bo_ref[...]


def _tc_mlp(u, i, c, dense, dense_W, dense_b, fc1_W, fc1_b,
            bn_gamma, bn_beta, fc2_W, fc2_b, out_W, out_b):
    B = u.shape[0]
    eu = u.shape[1]
    ec = 8
    cw = c.shape[1]
    nb = B // _BLK
    hdim = fc1_W.shape[0]

    def rows(bs):
        return pl.BlockSpec((_BLK, bs), lambda p, b: (b * (1 - p), 0))

    def full(shape):
        return pl.BlockSpec(shape, lambda p, b: (0,) * len(shape))

    return pl.pallas_call(
        functools.partial(_mlp_body, batch=float(B), eu=eu, ec=ec),
        grid=(2, nb),
        in_specs=[rows(eu), rows(eu), rows(cw), rows(2),
                  full((8, 2)), full((1, 8)),
                  full((hdim, 2 * eu + ec + 8)), full((1, hdim)),
                  full((1, hdim)), full((1, hdim)),
                  full((32, hdim)), full((1, 32)), full((32, 1)),
                  full((1, 1))],
        out_specs=pl.BlockSpec((_BLK, 1), lambda p, b: (b * p, 0)),
        out_shape=jax.ShapeDtypeStruct((B, 1), jnp.float32),
        scratch_shapes=[pltpu.VMEM((B, hdim), jnp.float32),
                        pltpu.VMEM((nb, hdim), jnp.float32),
                        pltpu.VMEM((nb, hdim), jnp.float32)],
    )(u, i, c, dense, dense_W, dense_b[None, :],
      fc1_W, fc1_b[None, :], bn_gamma[None, :], bn_beta[None, :],
      fc2_W, fc2_b[None, :], out_W.T, out_b[None, :])


def kernel(user, item, cat, dense, user_table, item_table, cat_table,
           dense_W, dense_b, fc1_W, fc1_b, bn_gamma, bn_beta,
           fc2_W, fc2_b, out_W, out_b):
    c = _sc_gather_cat(cat.astype(jnp.int32), cat_table)
    u = jnp.take(user_table, user, axis=0, mode="clip")
    i = jnp.take(item_table, item, axis=0, mode="clip")
    return _tc_mlp(u, i, c, dense, dense_W, dense_b, fc1_W, fc1_b,
                   bn_gamma, bn_beta, fc2_W, fc2_b, out_W, out_b)
